# Initial kernel scaffold; baseline (speedup 1.0000x reference)
import functools

import jax
import jax.numpy as jnp
from jax import lax
from jax.experimental import pallas as pl
from jax.experimental.pallas import tpu as pltpu
from jax.experimental.pallas import tpu_sc as plsc

N = 10000
C = 64
D = 256
E = 160000

NSC = 2
NS = 16
L = 16
EP = E // NS
K = 80
NB = EP // K
DH = D // NSC
ROWS_T = N // NS
HC = N * C
HCT = HC // NS


def _argmax_body(logits_ref, out_ref):
    out_ref[...] = jnp.argmax(logits_ref[...], axis=1).astype(jnp.int32)


def _edge_body(src_hbm, dst_hbm, cp_hbm, feats_hbm, za_hbm, zh_hbm,
               agg_out, hist_out,
               src_v, dst_v, cp_v, rows_v, keys_v, ones_v, agg_s, hist_s, sem):
    c = lax.axis_index("c")
    s = lax.axis_index("s")
    pltpu.sync_copy(src_hbm.at[pl.ds(s * EP, EP)], src_v)
    pltpu.sync_copy(dst_hbm.at[s], dst_v)
    pltpu.sync_copy(cp_hbm, cp_v)
    pltpu.sync_copy(za_hbm.at[s], agg_s.at[pl.ds(s * ROWS_T, ROWS_T)])
    pltpu.sync_copy(zh_hbm.at[pl.ds(s * HCT, HCT)], hist_s.at[pl.ds(s * HCT, HCT)])
    for i in range(K // L):
        ones_v[pl.ds(i * L, L)] = jnp.full((L,), 1.0, jnp.float32)
    plsc.subcore_barrier()

    def body(j, carry):
        pltpu.async_copy(
            feats_hbm.at[c].at[src_v.at[pl.ds(j * K, K)]], rows_v, sem).wait()
        pltpu.sync_copy(rows_v, agg_s.at[dst_v.at[j]], add=True)

        @pl.when((j % 2) == c)
        def _():
            for i in range(K // L):
                sv = src_v[pl.ds(j * K + i * L, L)]
                cls = plsc.load_gather(cp_v, [sv])
                dv = dst_v[j, pl.ds(i * L, L)]
                keys_v[pl.ds(i * L, L)] = dv * C + cls
            pltpu.sync_copy(ones_v, hist_s.at[keys_v], add=True)

        return carry

    lax.fori_loop(0, NB, body, 0)
    plsc.subcore_barrier()
    pltpu.sync_copy(agg_s.at[pl.ds(s * ROWS_T, ROWS_T)],
                    agg_out.at[c, pl.ds(s * ROWS_T, ROWS_T)])
    pltpu.sync_copy(hist_s.at[pl.ds(s * HCT, HCT)],
                    hist_out.at[c, pl.ds(s * HCT, HCT)])


def _gate_body(h_ref, cp_ref, oz_ref, t1_ref, t2_ref, gn_ref, z_ref):
    counts = h_ref[0] + h_ref[1]
    degs = jnp.maximum(jnp.sum(counts, axis=1, keepdims=True), 1.0)
    cpv = cp_ref[...]
    iot = lax.broadcasted_iota(jnp.int32, (N, C), 1)
    f1 = jnp.sum(jnp.where(iot == cpv, counts, 0.0), axis=1, keepdims=True) / degs
    p = jnp.maximum(counts / degs, 1e-5)
    f2 = -jnp.sum(p * jnp.log(p), axis=1, keepdims=True)

    def ln(x):
        m = jnp.mean(x)
        v = jnp.mean((x - m) ** 2)
        return (x - m) * lax.rsqrt(v + 1e-5)

    z = (jax.nn.sigmoid(-(ln(f1) - t1_ref[0])) *
         jax.nn.sigmoid(-(ln(f2) - t2_ref[0])))
    gate = jnp.minimum(oz_ref[...], z)
    gn_ref[...] = gate * lax.rsqrt(degs)
    z_ref[...] = z


BLK = 1000


def _update_body(feats_ref, a0_ref, a1_ref, gn_ref, out_ref):
    g = gn_ref[...]
    out_ref[:, :DH] = feats_ref[:, :DH] + g * a0_ref[...]
    out_ref[:, DH:] = feats_ref[:, DH:] + g * a1_ref[...]


def kernel(feats, logits, old_z, tau1, tau2, edge_index):
    src = edge_index[0]
    dst = edge_index[1]

    cp = pl.pallas_call(
        _argmax_body,
        out_shape=jax.ShapeDtypeStruct((N,), jnp.int32),
    )(logits)

    feats2 = feats.reshape(N, NSC, DH).transpose(1, 0, 2)
    dst3 = dst.reshape(NS, NB, K)
    za = jnp.zeros((NS, ROWS_T, DH), jnp.float32)
    zh = jnp.zeros((HC,), jnp.float32)

    mesh = plsc.VectorSubcoreMesh(core_axis_name="c", subcore_axis_name="s")
    edge_kernel = functools.partial(
        pl.kernel,
        out_type=[jax.ShapeDtypeStruct((NSC, N, DH), jnp.float32),
                  jax.ShapeDtypeStruct((NSC, HC), jnp.float32)],
        mesh=mesh,
        scratch_types=[
            pltpu.VMEM((EP,), jnp.int32),
            pltpu.VMEM((NB, K), jnp.int32),
            pltpu.VMEM((N,), jnp.int32),
            pltpu.VMEM((K, DH), jnp.float32),
            pltpu.VMEM((K,), jnp.int32),
            pltpu.VMEM((K,), jnp.float32),
            pltpu.VMEM_SHARED((N, DH), jnp.float32),
            pltpu.VMEM_SHARED((HC,), jnp.float32),
            pltpu.SemaphoreType.DMA,
        ],
    )(_edge_body)
    agg2, hist2 = edge_kernel(src, dst3, cp, feats2, za, zh)

    gn, z2 = pl.pallas_call(
        _gate_body,
        out_shape=[jax.ShapeDtypeStruct((N, 1), jnp.float32),
                   jax.ShapeDtypeStruct((N, 1), jnp.float32)],
    )(hist2.reshape(NSC, N, C), cp.reshape(N, 1), old_z.reshape(N, 1),
      tau1, tau2)

    new_h = pl.pallas_call(
        _update_body,
        grid=(N // BLK,),
        in_specs=[
            pl.BlockSpec((BLK, D), lambda i: (i, 0)),
            pl.BlockSpec((BLK, DH), lambda i: (i, 0)),
            pl.BlockSpec((BLK, DH), lambda i: (i, 0)),
            pl.BlockSpec((BLK, 1), lambda i: (i, 0)),
        ],
        out_specs=pl.BlockSpec((BLK, D), lambda i: (i, 0)),
        out_shape=jax.ShapeDtypeStruct((N, D), jnp.float32),
    )(feats, agg2[0], agg2[1], gn)

    return (new_h, z2.reshape(N))


# speedup vs baseline: 4.9069x; 4.9069x over previous
import functools

import jax
import jax.numpy as jnp
from jax import lax
from jax.experimental import pallas as pl
from jax.experimental.pallas import tpu as pltpu
from jax.experimental.pallas import tpu_sc as plsc

N = 10000
C = 64
D = 256
E = 160000

NSC = 2
NS = 16
L = 16
EP = E // NS
K = 80
NB = EP // K
DH = D // NSC
NH = N // NSC
HTRASH = NH * C
HSZ = NH * C + 8


def _argmax_body(logits_ref, out_ref):
    out_ref[...] = jnp.argmax(logits_ref[...], axis=1).astype(jnp.int32)


def _edge_body(src_hbm, dst_hbm, cp_hbm, feats_hbm, za_hbm, zh_hbm,
               agg_out, hist_out,
               src_r, dst_r, cls_r, keys_r, ones_v, rows_v, agg_s, hist_s,
               sem, sem2):
    c = lax.axis_index("c")
    s = lax.axis_index("s")

    @pl.when(s == 0)
    def _():
        pltpu.sync_copy(za_hbm, agg_s)

    @pl.when(s == 1)
    def _():
        pltpu.sync_copy(zh_hbm, hist_s)

    for i in range(K // L):
        ones_v[pl.ds(i * L, L)] = jnp.full((L,), 1.0, jnp.float32)
    nbase = c * NH
    plsc.subcore_barrier()

    def body(j, carry):
        base = s * EP + j * K
        pltpu.sync_copy(src_hbm.at[pl.ds(base, K)], src_r.at[0])
        pltpu.sync_copy(dst_hbm.at[pl.ds(base, K)], dst_r.at[0])
        pltpu.async_copy(feats_hbm.at[c].at[src_r.at[0]], rows_v, sem).wait()
        pltpu.sync_copy(rows_v, agg_s.at[dst_r.at[0]], add=True)

        pltpu.async_copy(cp_hbm.at[src_r.at[0]], cls_r.at[0], sem2).wait()
        for i in range(K // L):
            dv = dst_r[0, pl.ds(i * L, L)]
            cv = cls_r[0, pl.ds(i * L, L)]
            lk = (dv - nbase) * C + cv
            owned = (dv >= nbase) & (dv < nbase + NH)
            keys_r[0, pl.ds(i * L, L)] = jnp.where(
                owned, lk, jnp.full((L,), HTRASH, jnp.int32))
        pltpu.sync_copy(ones_v, hist_s.at[keys_r.at[0]], add=True)
        return carry

    lax.fori_loop(0, NB, body, 0)
    plsc.subcore_barrier()

    @pl.when(s == 0)
    def _():
        pltpu.sync_copy(agg_s, agg_out.at[c])

    @pl.when(s == 1)
    def _():
        pltpu.sync_copy(hist_s, hist_out.at[c, 0])


def _gate_body(h_ref, cp_ref, oz_ref, t1_ref, t2_ref, gn_ref, z_ref):
    counts = h_ref[...]
    degs = jnp.maximum(jnp.sum(counts, axis=1, keepdims=True), 1.0)
    cpv = cp_ref[...]
    iot = lax.broadcasted_iota(jnp.int32, (N, C), 1)
    f1 = jnp.sum(jnp.where(iot == cpv, counts, 0.0), axis=1, keepdims=True) / degs
    p = jnp.maximum(counts / degs, 1e-5)
    f2 = -jnp.sum(p * jnp.log(p), axis=1, keepdims=True)

    def ln(x):
        m = jnp.mean(x)
        v = jnp.mean((x - m) ** 2)
        return (x - m) * lax.rsqrt(v + 1e-5)

    z = (jax.nn.sigmoid(-(ln(f1) - t1_ref[0])) *
         jax.nn.sigmoid(-(ln(f2) - t2_ref[0])))
    gate = jnp.minimum(oz_ref[...], z)
    gn_ref[...] = gate * lax.rsqrt(degs)
    z_ref[...] = z


BLK = 1000


def _update_body(feats_ref, a0_ref, a1_ref, gn_ref, out_ref):
    g = gn_ref[...]
    out_ref[:, :DH] = feats_ref[:, :DH] + g * a0_ref[...]
    out_ref[:, DH:] = feats_ref[:, DH:] + g * a1_ref[...]


def kernel(feats, logits, old_z, tau1, tau2, edge_index):
    src = edge_index[0]
    dst = edge_index[1]

    cp = pl.pallas_call(
        _argmax_body,
        out_shape=jax.ShapeDtypeStruct((N,), jnp.int32),
    )(logits)

    feats2 = feats.reshape(N, NSC, DH).transpose(1, 0, 2)
    za = jnp.zeros((N, DH), jnp.float32)
    zh = jnp.zeros((HSZ,), jnp.float32)

    mesh = plsc.VectorSubcoreMesh(core_axis_name="c", subcore_axis_name="s")
    edge_kernel = functools.partial(
        pl.kernel,
        out_type=[jax.ShapeDtypeStruct((NSC, N, DH), jnp.float32),
                  jax.ShapeDtypeStruct((NSC, 1, HSZ), jnp.float32)],
        mesh=mesh,
        scratch_types=[
            pltpu.VMEM((1, K), jnp.int32),
            pltpu.VMEM((1, K), jnp.int32),
            pltpu.VMEM((1, K), jnp.int32),
            pltpu.VMEM((1, K), jnp.int32),
            pltpu.VMEM((K,), jnp.float32),
            pltpu.VMEM((K, DH), jnp.float32),
            pltpu.VMEM_SHARED((N, DH), jnp.float32),
            pltpu.VMEM_SHARED((HSZ,), jnp.float32),
            pltpu.SemaphoreType.DMA,
            pltpu.SemaphoreType.DMA,
        ],
        compiler_params=pltpu.CompilerParams(needs_layout_passes=False),
    )(_edge_body)
    agg2, hist2 = edge_kernel(src, dst, cp, feats2, za, zh)

    counts = jnp.concatenate(
        [hist2[0, 0, :NH * C].reshape(NH, C),
         hist2[1, 0, :NH * C].reshape(NH, C)], axis=0)

    gn, z2 = pl.pallas_call(
        _gate_body,
        out_shape=[jax.ShapeDtypeStruct((N, 1), jnp.float32),
                   jax.ShapeDtypeStruct((N, 1), jnp.float32)],
    )(counts, cp.reshape(N, 1), old_z.reshape(N, 1), tau1, tau2)

    new_h = pl.pallas_call(
        _update_body,
        grid=(N // BLK,),
        in_specs=[
            pl.BlockSpec((BLK, D), lambda i: (i, 0)),
            pl.BlockSpec((BLK, DH), lambda i: (i, 0)),
            pl.BlockSpec((BLK, DH), lambda i: (i, 0)),
            pl.BlockSpec((BLK, 1), lambda i: (i, 0)),
        ],
        out_specs=pl.BlockSpec((BLK, D), lambda i: (i, 0)),
        out_shape=jax.ShapeDtypeStruct((N, D), jnp.float32),
    )(feats, agg2[0], agg2[1], gn)

    return (new_h, z2.reshape(N))


# speedup vs baseline: 9.2757x; 1.8903x over previous
import functools

import jax
import jax.numpy as jnp
from jax import lax
from jax.experimental import pallas as pl
from jax.experimental.pallas import tpu as pltpu
from jax.experimental.pallas import tpu_sc as plsc

N = 10000
C = 64
D = 256
E = 160000

NSC = 2
NS = 16
L = 16
EP = E // NS
K = 80
NB = EP // K
DH = D // NSC
NH = N // NSC
HTRASH = NH * C
HSZ = NH * C + 8


def _argmax_body(logits_ref, out_ref):
    out_ref[...] = jnp.argmax(logits_ref[...], axis=1).astype(jnp.int32)


def _edge_body(src_hbm, dst_hbm, cp_hbm, feats_hbm, za_hbm, zh_hbm,
               agg_out, hist_out,
               src_r, dst_r, cls_r, keys_r, ones_v, rows_v, agg_s, hist_s,
               sem_a, sem_b, sem_g, sem_c):
    c = lax.axis_index("c")
    s = lax.axis_index("s")

    @pl.when(s == 0)
    def _():
        pltpu.sync_copy(za_hbm, agg_s)

    @pl.when(s == 1)
    def _():
        pltpu.sync_copy(zh_hbm, hist_s)

    for i in range(K // L):
        ones_v[pl.ds(i * L, L)] = jnp.full((L,), 1.0, jnp.float32)
    nbase = c * NH
    plsc.subcore_barrier()

    ebase = s * EP

    def issue_idx(j, slot):
        pltpu.async_copy(src_hbm.at[pl.ds(ebase + j * K, K)], src_r.at[slot],
                         sem_a)
        pltpu.async_copy(dst_hbm.at[pl.ds(ebase + j * K, K)], dst_r.at[slot],
                         sem_b)

    def wait_idx(slot):
        pltpu.make_async_copy(src_hbm.at[pl.ds(ebase, K)], src_r.at[slot],
                              sem_a).wait()
        pltpu.make_async_copy(dst_hbm.at[pl.ds(ebase, K)], dst_r.at[slot],
                              sem_b).wait()

    def issue_gathers(islot, rslot):
        pltpu.async_copy(feats_hbm.at[c].at[src_r.at[islot]],
                         rows_v.at[rslot], sem_g)
        pltpu.async_copy(cp_hbm.at[src_r.at[islot]], cls_r.at[rslot], sem_c)

    pltpu.sync_copy(src_hbm.at[pl.ds(ebase, K)], src_r.at[0])
    pltpu.sync_copy(dst_hbm.at[pl.ds(ebase, K)], dst_r.at[0])
    issue_gathers(0, 0)
    issue_idx(1, 1)

    def body(j, carry):
        r2 = lax.rem(j, 2)
        r2n = lax.rem(j + 1, 2)
        r3 = lax.rem(j, 3)
        r3n = lax.rem(j + 1, 3)
        r3nn = lax.rem(j + 2, 3)

        @pl.when(j + 1 < NB)
        def _():
            wait_idx(r3n)
            issue_gathers(r3n, r2n)

        @pl.when(j + 2 < NB)
        def _():
            issue_idx(j + 2, r3nn)

        pltpu.make_async_copy(feats_hbm.at[c].at[src_r.at[r3]],
                              rows_v.at[r2], sem_g).wait()
        pltpu.sync_copy(rows_v.at[r2], agg_s.at[dst_r.at[r3]], add=True)

        pltpu.make_async_copy(cp_hbm.at[src_r.at[r3]], cls_r.at[r2],
                              sem_c).wait()
        for i in range(K // L):
            dv = dst_r[r3, pl.ds(i * L, L)]
            cv = cls_r[r2, pl.ds(i * L, L)]
            lk = (dv - nbase) * C + cv
            owned = (dv >= nbase) & (dv < nbase + NH)
            keys_r[0, pl.ds(i * L, L)] = jnp.where(
                owned, lk, jnp.full((L,), HTRASH, jnp.int32))
        pltpu.sync_copy(ones_v, hist_s.at[keys_r.at[0]], add=True)
        return carry

    lax.fori_loop(0, NB, body, 0)
    plsc.subcore_barrier()

    @pl.when(s == 0)
    def _():
        pltpu.sync_copy(agg_s, agg_out.at[c])

    @pl.when(s == 1)
    def _():
        pltpu.sync_copy(hist_s, hist_out.at[c, 0])


def _gate_body(h_ref, cp_ref, oz_ref, t1_ref, t2_ref, gn_ref, z_ref):
    counts = h_ref[...]
    degs = jnp.maximum(jnp.sum(counts, axis=1, keepdims=True), 1.0)
    cpv = cp_ref[...]
    iot = lax.broadcasted_iota(jnp.int32, (N, C), 1)
    f1 = jnp.sum(jnp.where(iot == cpv, counts, 0.0), axis=1, keepdims=True) / degs
    p = jnp.maximum(counts / degs, 1e-5)
    f2 = -jnp.sum(p * jnp.log(p), axis=1, keepdims=True)

    def ln(x):
        m = jnp.mean(x)
        v = jnp.mean((x - m) ** 2)
        return (x - m) * lax.rsqrt(v + 1e-5)

    z = (jax.nn.sigmoid(-(ln(f1) - t1_ref[0])) *
         jax.nn.sigmoid(-(ln(f2) - t2_ref[0])))
    gate = jnp.minimum(oz_ref[...], z)
    gn_ref[...] = gate * lax.rsqrt(degs)
    z_ref[...] = z


BLK = 1000


def _update_body(feats_ref, a0_ref, a1_ref, gn_ref, out_ref):
    g = gn_ref[...]
    out_ref[:, :DH] = feats_ref[:, :DH] + g * a0_ref[...]
    out_ref[:, DH:] = feats_ref[:, DH:] + g * a1_ref[...]


def kernel(feats, logits, old_z, tau1, tau2, edge_index):
    src = edge_index[0]
    dst = edge_index[1]

    cp = pl.pallas_call(
        _argmax_body,
        out_shape=jax.ShapeDtypeStruct((N,), jnp.int32),
    )(logits)

    feats2 = feats.reshape(N, NSC, DH).transpose(1, 0, 2)
    za = jnp.zeros((N, DH), jnp.float32)
    zh = jnp.zeros((HSZ,), jnp.float32)

    mesh = plsc.VectorSubcoreMesh(core_axis_name="c", subcore_axis_name="s")
    edge_kernel = functools.partial(
        pl.kernel,
        out_type=[jax.ShapeDtypeStruct((NSC, N, DH), jnp.float32),
                  jax.ShapeDtypeStruct((NSC, 1, HSZ), jnp.float32)],
        mesh=mesh,
        scratch_types=[
            pltpu.VMEM((3, K), jnp.int32),
            pltpu.VMEM((3, K), jnp.int32),
            pltpu.VMEM((2, K), jnp.int32),
            pltpu.VMEM((1, K), jnp.int32),
            pltpu.VMEM((K,), jnp.float32),
            pltpu.VMEM((2, K, DH), jnp.float32),
            pltpu.VMEM_SHARED((N, DH), jnp.float32),
            pltpu.VMEM_SHARED((HSZ,), jnp.float32),
            pltpu.SemaphoreType.DMA,
            pltpu.SemaphoreType.DMA,
            pltpu.SemaphoreType.DMA,
            pltpu.SemaphoreType.DMA,
        ],
        compiler_params=pltpu.CompilerParams(needs_layout_passes=False),
    )(_edge_body)
    agg2, hist2 = edge_kernel(src, dst, cp, feats2, za, zh)

    counts = jnp.concatenate(
        [hist2[0, 0, :NH * C].reshape(NH, C),
         hist2[1, 0, :NH * C].reshape(NH, C)], axis=0)

    gn, z2 = pl.pallas_call(
        _gate_body,
        out_shape=[jax.ShapeDtypeStruct((N, 1), jnp.float32),
                   jax.ShapeDtypeStruct((N, 1), jnp.float32)],
    )(counts, cp.reshape(N, 1), old_z.reshape(N, 1), tau1, tau2)

    new_h = pl.pallas_call(
        _update_body,
        grid=(N // BLK,),
        in_specs=[
            pl.BlockSpec((BLK, D), lambda i: (i, 0)),
            pl.BlockSpec((BLK, DH), lambda i: (i, 0)),
            pl.BlockSpec((BLK, DH), lambda i: (i, 0)),
            pl.BlockSpec((BLK, 1), lambda i: (i, 0)),
        ],
        out_specs=pl.BlockSpec((BLK, D), lambda i: (i, 0)),
        out_shape=jax.ShapeDtypeStruct((N, D), jnp.float32),
    )(feats, agg2[0], agg2[1], gn)

    return (new_h, z2.reshape(N))


# speedup vs baseline: 9.4716x; 1.0211x over previous
import functools

import jax
import jax.numpy as jnp
from jax import lax
from jax.experimental import pallas as pl
from jax.experimental.pallas import tpu as pltpu
from jax.experimental.pallas import tpu_sc as plsc

N = 10000
C = 64
D = 256
E = 160000

NSC = 2
NS = 16
L = 16
EP = E // NS
K = 80
NB = EP // K
DH = D // NSC
NH = N // NSC
HTRASH = NH * C
HSZ = NH * C + 8


def _argmax_body(logits_ref, out_ref):
    out_ref[...] = jnp.argmax(logits_ref[...], axis=1).astype(jnp.int32)


def _edge_body(src_hbm, dst_hbm, cp_hbm, feats_hbm, za_hbm, zh_hbm,
               agg_out, hist_out,
               src_r, dst_r, cls_r, keys_r, ones_v, rows_v, agg_s, hist_s,
               sem_a, sem_b, sem_g, sem_c, sem_s, sem_h):
    c = lax.axis_index("c")
    s = lax.axis_index("s")

    @pl.when(s == 0)
    def _():
        pltpu.sync_copy(za_hbm, agg_s)

    @pl.when(s == 1)
    def _():
        pltpu.sync_copy(zh_hbm, hist_s)

    for i in range(K // L):
        ones_v[pl.ds(i * L, L)] = jnp.full((L,), 1.0, jnp.float32)
    nbase = c * NH
    plsc.subcore_barrier()

    ebase = s * EP

    def issue_idx(j, slot):
        pltpu.async_copy(src_hbm.at[pl.ds(ebase + j * K, K)], src_r.at[slot],
                         sem_a)
        pltpu.async_copy(dst_hbm.at[pl.ds(ebase + j * K, K)], dst_r.at[slot],
                         sem_b)

    def wait_idx(slot):
        pltpu.make_async_copy(src_hbm.at[pl.ds(ebase, K)], src_r.at[slot],
                              sem_a).wait()
        pltpu.make_async_copy(dst_hbm.at[pl.ds(ebase, K)], dst_r.at[slot],
                              sem_b).wait()

    def issue_gathers(islot, rslot):
        pltpu.async_copy(feats_hbm.at[c].at[src_r.at[islot]],
                         rows_v.at[rslot], sem_g)
        pltpu.async_copy(cp_hbm.at[src_r.at[islot]], cls_r.at[rslot], sem_c)

    pltpu.sync_copy(src_hbm.at[pl.ds(ebase, K)], src_r.at[0])
    pltpu.sync_copy(dst_hbm.at[pl.ds(ebase, K)], dst_r.at[0])
    issue_gathers(0, 0)
    issue_idx(1, 1)

    def wait_scat(r2, r3):
        pltpu.make_async_copy(rows_v.at[r2], agg_s.at[dst_r.at[r3]],
                              sem_s).wait()

    def wait_hist(slot):
        pltpu.make_async_copy(ones_v, hist_s.at[keys_r.at[slot]],
                              sem_h).wait()

    def body(j, carry):
        r2 = lax.rem(j, 2)
        r2n = lax.rem(j + 1, 2)
        r3 = lax.rem(j, 3)
        r3n = lax.rem(j + 1, 3)
        r3nn = lax.rem(j + 2, 3)

        @pl.when(j + 1 < NB)
        def _():
            wait_idx(r3n)

        @pl.when(j >= 1)
        def _():
            wait_scat(r2n, lax.rem(j + 2, 3))

        @pl.when(j + 1 < NB)
        def _():
            issue_gathers(r3n, r2n)

        @pl.when(j + 2 < NB)
        def _():
            issue_idx(j + 2, r3nn)

        pltpu.make_async_copy(feats_hbm.at[c].at[src_r.at[r3]],
                              rows_v.at[r2], sem_g).wait()
        pltpu.async_copy(rows_v.at[r2], agg_s.at[dst_r.at[r3]], sem_s,
                         add=True)

        pltpu.make_async_copy(cp_hbm.at[src_r.at[r3]], cls_r.at[r2],
                              sem_c).wait()

        @pl.when(j >= 2)
        def _():
            wait_hist(r2)

        for i in range(K // L):
            dv = dst_r[r3, pl.ds(i * L, L)]
            cv = cls_r[r2, pl.ds(i * L, L)]
            lk = (dv - nbase) * C + cv
            owned = (dv >= nbase) & (dv < nbase + NH)
            keys_r[r2, pl.ds(i * L, L)] = jnp.where(
                owned, lk, jnp.full((L,), HTRASH, jnp.int32))
        pltpu.async_copy(ones_v, hist_s.at[keys_r.at[r2]], sem_h, add=True)
        return carry

    lax.fori_loop(0, NB, body, 0)
    wait_scat(lax.rem(NB - 1, 2), lax.rem(NB - 1, 3))
    wait_hist(lax.rem(NB - 2, 2))
    wait_hist(lax.rem(NB - 1, 2))
    plsc.subcore_barrier()

    @pl.when(s == 0)
    def _():
        pltpu.sync_copy(agg_s, agg_out.at[c])

    @pl.when(s == 1)
    def _():
        pltpu.sync_copy(hist_s, hist_out.at[c, 0])


BLK = 1000


def _finish_body(h_ref, cp_ref, oz_ref, t1_ref, t2_ref,
                 feats_ref, a0_ref, a1_ref, out_ref, z_ref, gn_s):
    i = pl.program_id(0)

    @pl.when(i == 0)
    def _():
        counts = h_ref[...]
        degs = jnp.maximum(jnp.sum(counts, axis=1, keepdims=True), 1.0)
        cpv = cp_ref[...]
        iot = lax.broadcasted_iota(jnp.int32, (N, C), 1)
        f1 = (jnp.sum(jnp.where(iot == cpv, counts, 0.0), axis=1,
                      keepdims=True) / degs)
        p = jnp.maximum(counts / degs, 1e-5)
        f2 = -jnp.sum(p * jnp.log(p), axis=1, keepdims=True)

        def ln(x):
            m = jnp.mean(x)
            v = jnp.mean((x - m) ** 2)
            return (x - m) * lax.rsqrt(v + 1e-5)

        z = (jax.nn.sigmoid(-(ln(f1) - t1_ref[0])) *
             jax.nn.sigmoid(-(ln(f2) - t2_ref[0])))
        gate = jnp.minimum(oz_ref[...], z)
        gn_s[...] = gate * lax.rsqrt(degs)
        z_ref[...] = z

    g = gn_s[pl.ds(i * BLK, BLK), :]
    out_ref[:, :DH] = feats_ref[:, :DH] + g * a0_ref[...]
    out_ref[:, DH:] = feats_ref[:, DH:] + g * a1_ref[...]


def kernel(feats, logits, old_z, tau1, tau2, edge_index):
    src = edge_index[0]
    dst = edge_index[1]

    cp = pl.pallas_call(
        _argmax_body,
        out_shape=jax.ShapeDtypeStruct((N,), jnp.int32),
    )(logits)

    feats2 = feats.reshape(N, NSC, DH).transpose(1, 0, 2)
    za = jnp.zeros((N, DH), jnp.float32)
    zh = jnp.zeros((HSZ,), jnp.float32)

    mesh = plsc.VectorSubcoreMesh(core_axis_name="c", subcore_axis_name="s")
    edge_kernel = functools.partial(
        pl.kernel,
        out_type=[jax.ShapeDtypeStruct((NSC, N, DH), jnp.float32),
                  jax.ShapeDtypeStruct((NSC, 1, HSZ), jnp.float32)],
        mesh=mesh,
        scratch_types=[
            pltpu.VMEM((3, K), jnp.int32),
            pltpu.VMEM((3, K), jnp.int32),
            pltpu.VMEM((2, K), jnp.int32),
            pltpu.VMEM((2, K), jnp.int32),
            pltpu.VMEM((K,), jnp.float32),
            pltpu.VMEM((2, K, DH), jnp.float32),
            pltpu.VMEM_SHARED((N, DH), jnp.float32),
            pltpu.VMEM_SHARED((HSZ,), jnp.float32),
            pltpu.SemaphoreType.DMA,
            pltpu.SemaphoreType.DMA,
            pltpu.SemaphoreType.DMA,
            pltpu.SemaphoreType.DMA,
            pltpu.SemaphoreType.DMA,
            pltpu.SemaphoreType.DMA,
        ],
        compiler_params=pltpu.CompilerParams(needs_layout_passes=False),
    )(_edge_body)
    agg2, hist2 = edge_kernel(src, dst, cp, feats2, za, zh)

    counts = jnp.concatenate(
        [hist2[0, 0, :NH * C].reshape(NH, C),
         hist2[1, 0, :NH * C].reshape(NH, C)], axis=0)

    new_h, z2 = pl.pallas_call(
        _finish_body,
        grid=(N // BLK,),
        in_specs=[
            pl.BlockSpec((N, C), lambda i: (0, 0)),
            pl.BlockSpec((N, 1), lambda i: (0, 0)),
            pl.BlockSpec((N, 1), lambda i: (0, 0)),
            pl.BlockSpec((1,), lambda i: (0,)),
            pl.BlockSpec((1,), lambda i: (0,)),
            pl.BlockSpec((BLK, D), lambda i: (i, 0)),
            pl.BlockSpec((BLK, DH), lambda i: (i, 0)),
            pl.BlockSpec((BLK, DH), lambda i: (i, 0)),
        ],
        out_specs=[pl.BlockSpec((BLK, D), lambda i: (i, 0)),
                   pl.BlockSpec((N, 1), lambda i: (0, 0))],
        out_shape=[jax.ShapeDtypeStruct((N, D), jnp.float32),
                   jax.ShapeDtypeStruct((N, 1), jnp.float32)],
        scratch_shapes=[pltpu.VMEM((N, 1), jnp.float32)],
    )(counts, cp.reshape(N, 1), old_z.reshape(N, 1), tau1, tau2,
      feats, agg2[0], agg2[1])

    return (new_h, z2.reshape(N))
